# Initial kernel scaffold; baseline (speedup 1.0000x reference)
import jax
import jax.numpy as jnp
from jax.experimental import pallas as pl


def kernel(mix, falsetto, breathy, pharyngeal, glissando, vibrato, mix_emb, falsetto_emb, breathy_emb, pharyngeal_emb, glissando_emb, vibrato_emb):
    raise NotImplementedError("write your pallas kernel here")


# speedup vs baseline: 9.2514x; 9.2514x over previous
import math

import jax
import jax.numpy as jnp
from jax.experimental import pallas as pl
from jax.experimental.pallas import tpu as pltpu

H = 256
TB = 2048


def _body(i0, i1, i2, i3, i4, i5, e0, e1, e2, e3, e4, e5, out_ref):
    s = math.sqrt(H)
    es = [e0[...], e1[...], e2[...], e3[...], e4[...], e5[...]]
    deltas = [(e[1:2, :] - e[0:1, :]) * s for e in es]
    base = (es[0][0:1] + es[1][0:1] + es[2][0:1]
            + es[3][0:1] + es[4][0:1] + es[5][0:1]) * s
    d = jnp.concatenate(deltas + [base, jnp.zeros_like(base)], axis=0)
    cols = [r[...].astype(jnp.float32) for r in (i0, i1, i2, i3, i4, i5)]
    ones = jnp.ones_like(cols[0])
    x = jnp.concatenate(cols + [ones, jnp.zeros_like(ones)], axis=1)
    out_ref[...] = jnp.dot(x, d, preferred_element_type=jnp.float32)


def kernel(mix, falsetto, breathy, pharyngeal, glissando, vibrato,
           mix_emb, falsetto_emb, breathy_emb, pharyngeal_emb,
           glissando_emb, vibrato_emb):
    B, L = mix.shape
    n = B * L
    idxs = [a.reshape(n, 1) for a in
            (mix, falsetto, breathy, pharyngeal, glissando, vibrato)]
    embs = (mix_emb, falsetto_emb, breathy_emb, pharyngeal_emb,
            glissando_emb, vibrato_emb)
    grid = (n // TB,)
    idx_spec = pl.BlockSpec((TB, 1), lambda i: (i, 0))
    emb_spec = pl.BlockSpec((2, H), lambda i: (0, 0))
    out = pl.pallas_call(
        _body,
        grid=grid,
        in_specs=[idx_spec] * 6 + [emb_spec] * 6,
        out_specs=pl.BlockSpec((TB, H), lambda i: (i, 0)),
        out_shape=jax.ShapeDtypeStruct((n, H), jnp.float32),
    )(*idxs, *embs)
    return out.reshape(B, L, H)


# speedup vs baseline: 10.8816x; 1.1762x over previous
import math

import jax
import jax.numpy as jnp
from jax import lax
from jax.experimental import pallas as pl
from jax.experimental.pallas import tpu as pltpu

H = 256
TPR = 16
F = 8
TBR = 128
NC = TPR * H


def _body(xi_ref, e0, e1, e2, e3, e4, e5, out_ref, dbd_ref):
    @pl.when(pl.program_id(0) == 0)
    def _init():
        s = math.sqrt(H)
        es = [e0[...], e1[...], e2[...], e3[...], e4[...], e5[...]]
        deltas = [(e[1:2, :] - e[0:1, :]) * s for e in es]
        base = (es[0][0:1] + es[1][0:1] + es[2][0:1]
                + es[3][0:1] + es[4][0:1] + es[5][0:1]) * s
        d = jnp.concatenate(deltas + [base, jnp.zeros_like(base)], axis=0)
        dt = jnp.tile(d, (TPR, TPR))
        j = lax.broadcasted_iota(jnp.int32, (TPR * F, NC), 0)
        c = lax.broadcasted_iota(jnp.int32, (TPR * F, NC), 1)
        dbd_ref[...] = jnp.where((j // F) == (c // H), dt, 0.0)

    x = xi_ref[...].astype(jnp.float32)
    out_ref[...] = jnp.dot(x, dbd_ref[...],
                           preferred_element_type=jnp.float32)


def kernel(mix, falsetto, breathy, pharyngeal, glissando, vibrato,
           mix_emb, falsetto_emb, breathy_emb, pharyngeal_emb,
           glissando_emb, vibrato_emb):
    B, L = mix.shape
    n = B * L
    nr = n // TPR
    ones = jnp.ones((B, L), jnp.int32)
    xi = jnp.stack([mix, falsetto, breathy, pharyngeal, glissando, vibrato,
                    ones, jnp.zeros((B, L), jnp.int32)], axis=-1)
    xi = xi.reshape(nr, TPR * F)
    embs = (mix_emb, falsetto_emb, breathy_emb, pharyngeal_emb,
            glissando_emb, vibrato_emb)
    grid = (nr // TBR,)
    emb_spec = pl.BlockSpec((2, H), lambda i: (0, 0))
    out = pl.pallas_call(
        _body,
        grid=grid,
        in_specs=[pl.BlockSpec((TBR, TPR * F), lambda i: (i, 0))]
        + [emb_spec] * 6,
        out_specs=pl.BlockSpec((TBR, NC), lambda i: (i, 0)),
        out_shape=jax.ShapeDtypeStruct((nr, NC), jnp.float32),
        scratch_shapes=[pltpu.VMEM((TPR * F, NC), jnp.float32)],
    )(xi, *embs)
    return out.reshape(B, L, H)


# speedup vs baseline: 11.5704x; 1.0633x over previous
import math

import jax
import jax.numpy as jnp
from jax import lax
from jax.experimental import pallas as pl
from jax.experimental.pallas import tpu as pltpu

H = 256
TPR = 16
F = 8
TBR = 256
NC = TPR * H


def _body(xi_ref, e0, e1, e2, e3, e4, e5, out_ref, dbd_ref):
    @pl.when(pl.program_id(0) == 0)
    def _init():
        s = math.sqrt(H)
        es = [e0[...], e1[...], e2[...], e3[...], e4[...], e5[...]]
        deltas = [(e[1:2, :] - e[0:1, :]) * s for e in es]
        base = (es[0][0:1] + es[1][0:1] + es[2][0:1]
                + es[3][0:1] + es[4][0:1] + es[5][0:1]) * s
        d = jnp.concatenate(deltas + [base, jnp.zeros_like(base)], axis=0)
        dt = jnp.tile(d, (TPR, TPR))
        j = lax.broadcasted_iota(jnp.int32, (TPR * F, NC), 0)
        c = lax.broadcasted_iota(jnp.int32, (TPR * F, NC), 1)
        dbd_ref[...] = jnp.where((j // F) == (c // H), dt, 0.0)

    x = xi_ref[...].astype(jnp.float32)
    out_ref[...] = jnp.dot(x, dbd_ref[...],
                           preferred_element_type=jnp.float32)


def kernel(mix, falsetto, breathy, pharyngeal, glissando, vibrato,
           mix_emb, falsetto_emb, breathy_emb, pharyngeal_emb,
           glissando_emb, vibrato_emb):
    B, L = mix.shape
    n = B * L
    nr = n // TPR
    ones = jnp.ones((B, L), jnp.int32)
    xi = jnp.stack([mix, falsetto, breathy, pharyngeal, glissando, vibrato,
                    ones, jnp.zeros((B, L), jnp.int32)], axis=-1)
    xi = xi.reshape(nr, TPR * F)
    embs = (mix_emb, falsetto_emb, breathy_emb, pharyngeal_emb,
            glissando_emb, vibrato_emb)
    grid = (nr // TBR,)
    emb_spec = pl.BlockSpec((2, H), lambda i: (0, 0))
    out = pl.pallas_call(
        _body,
        grid=grid,
        in_specs=[pl.BlockSpec((TBR, TPR * F), lambda i: (i, 0))]
        + [emb_spec] * 6,
        out_specs=pl.BlockSpec((TBR, NC), lambda i: (i, 0)),
        out_shape=jax.ShapeDtypeStruct((nr, NC), jnp.float32),
        scratch_shapes=[pltpu.VMEM((TPR * F, NC), jnp.float32)],
    )(xi, *embs)
    return out.reshape(B, L, H)


# speedup vs baseline: 11.8538x; 1.0245x over previous
import math

import jax
import jax.numpy as jnp
from jax import lax
from jax.experimental import pallas as pl
from jax.experimental.pallas import tpu as pltpu

H = 256
TPR = 16
F = 8
TBR = 512
NC = TPR * H


def _body(xi_ref, e0, e1, e2, e3, e4, e5, out_ref, dbd_ref):
    @pl.when(pl.program_id(0) == 0)
    def _init():
        s = math.sqrt(H)
        es = [e0[...], e1[...], e2[...], e3[...], e4[...], e5[...]]
        deltas = [(e[1:2, :] - e[0:1, :]) * s for e in es]
        base = (es[0][0:1] + es[1][0:1] + es[2][0:1]
                + es[3][0:1] + es[4][0:1] + es[5][0:1]) * s
        d = jnp.concatenate(deltas + [base, jnp.zeros_like(base)], axis=0)
        dt = jnp.tile(d, (TPR, TPR))
        j = lax.broadcasted_iota(jnp.int32, (TPR * F, NC), 0)
        c = lax.broadcasted_iota(jnp.int32, (TPR * F, NC), 1)
        dbd_ref[...] = jnp.where((j // F) == (c // H), dt, 0.0)

    x = xi_ref[...].astype(jnp.float32)
    out_ref[...] = jnp.dot(x, dbd_ref[...],
                           preferred_element_type=jnp.float32)


def kernel(mix, falsetto, breathy, pharyngeal, glissando, vibrato,
           mix_emb, falsetto_emb, breathy_emb, pharyngeal_emb,
           glissando_emb, vibrato_emb):
    B, L = mix.shape
    n = B * L
    nr = n // TPR
    ones = jnp.ones((B, L), jnp.int32)
    xi = jnp.stack([mix, falsetto, breathy, pharyngeal, glissando, vibrato,
                    ones, jnp.zeros((B, L), jnp.int32)], axis=-1)
    xi = xi.reshape(nr, TPR * F)
    embs = (mix_emb, falsetto_emb, breathy_emb, pharyngeal_emb,
            glissando_emb, vibrato_emb)
    grid = (nr // TBR,)
    emb_spec = pl.BlockSpec((2, H), lambda i: (0, 0))
    out = pl.pallas_call(
        _body,
        grid=grid,
        in_specs=[pl.BlockSpec((TBR, TPR * F), lambda i: (i, 0))]
        + [emb_spec] * 6,
        out_specs=pl.BlockSpec((TBR, NC), lambda i: (i, 0)),
        out_shape=jax.ShapeDtypeStruct((nr, NC), jnp.float32),
        scratch_shapes=[pltpu.VMEM((TPR * F, NC), jnp.float32)],
    )(xi, *embs)
    return out.reshape(B, L, H)
